# Initial kernel scaffold; baseline (speedup 1.0000x reference)
import jax
import jax.numpy as jnp
from jax.experimental import pallas as pl


def kernel(src_masks, pred_ious, targets_batch):
    raise NotImplementedError("write your pallas kernel here")


# speedup vs baseline: 1.7284x; 1.7284x over previous
import functools

import jax
import jax.numpy as jnp
import numpy as np
from jax import lax
from jax.experimental import pallas as pl
from jax.experimental.pallas import tpu as pltpu
from jax.experimental.pallas import tpu_sc as plsc

N, M, H, W = 16, 3, 512, 512
HW = H * W
K = 2048
K_OVER = 6144
K_IMP = 1536
K_REM = 512
K_ALL = K_OVER + K_REM
FOCAL_ALPHA = 0.25
FOCAL_GAMMA = 2.0
W_MASK = 20.0
W_DICE = 1.0
W_IOU = 1.0
NUM_OBJECTS = 16.0

SRC_TOT = M * N * K_ALL
TGT_TOT = N * K_ALL


@functools.lru_cache(maxsize=1)
def _point_indices():
    key = jax.random.key(1234)
    ky, kx, ky2, kx2 = jax.random.split(key, 4)
    y_rand = np.asarray(jax.random.randint(ky, (N, K_OVER), 0, H))
    x_rand = np.asarray(jax.random.randint(kx, (N, K_OVER), 0, W))
    y_rest = np.asarray(jax.random.randint(ky2, (N, K_REM), 0, H))
    x_rest = np.asarray(jax.random.randint(kx2, (N, K_REM), 0, W))
    lin = np.concatenate([y_rand * W + x_rand, y_rest * W + x_rest], axis=1)
    lin = lin.astype(np.int32)
    n_ids = np.arange(N, dtype=np.int32)[:, None]
    src_idx = np.empty((M, N, K_ALL), dtype=np.int32)
    for c in range(M):
        src_idx[c] = (n_ids * M + c) * HW + lin
    tgt_idx = (n_ids * HW + lin).astype(np.int32)
    return src_idx.reshape(-1), tgt_idx.reshape(-1)


_SRC_IDX_NP, _TGT_IDX_NP = _point_indices()


def _build_sc_gather():
    info = plsc.get_sparse_core_info()
    nw = info.num_cores * info.num_subcores
    src_per = SRC_TOT // nw
    tgt_per = TGT_TOT // nw
    assert SRC_TOT % nw == 0 and TGT_TOT % nw == 0
    assert src_per % 8 == 0 and tgt_per % 8 == 0
    mesh = plsc.VectorSubcoreMesh(core_axis_name="c", subcore_axis_name="s")

    @functools.partial(
        pl.kernel,
        out_type=[
            jax.ShapeDtypeStruct((SRC_TOT,), jnp.float32),
            jax.ShapeDtypeStruct((TGT_TOT,), jnp.float32),
        ],
        mesh=mesh,
        scratch_types=[
            pltpu.VMEM((src_per,), jnp.int32),
            pltpu.VMEM((src_per,), jnp.float32),
            pltpu.VMEM((tgt_per,), jnp.int32),
            pltpu.VMEM((tgt_per,), jnp.float32),
            pltpu.SemaphoreType.DMA,
            pltpu.SemaphoreType.DMA,
        ],
    )
    def gather(src_flat, src_idx, tgt_flat, tgt_idx, src_out, tgt_out,
               sidx_v, sval_v, tidx_v, tval_v, sem_s, sem_t):
        wid = lax.axis_index("s") * info.num_cores + lax.axis_index("c")
        sbase = wid * src_per
        tbase = wid * tgt_per
        pltpu.sync_copy(src_idx.at[pl.ds(sbase, src_per)], sidx_v)
        pltpu.sync_copy(tgt_idx.at[pl.ds(tbase, tgt_per)], tidx_v)
        cp_s = pltpu.async_copy(src_flat.at[sidx_v], sval_v, sem_s)
        cp_t = pltpu.async_copy(tgt_flat.at[tidx_v], tval_v, sem_t)
        cp_s.wait()
        cp_t.wait()
        pltpu.sync_copy(sval_v, src_out.at[pl.ds(sbase, src_per)])
        pltpu.sync_copy(tval_v, tgt_out.at[pl.ds(tbase, tgt_per)])

    return gather


BH = 128
NBLK = H // BH
GRID = N * NBLK


def _tc_body(src_ref, tgt_ref, sv_ref, tv_ref, piou_ref, out_ref,
             acc_i, acc_u):
    i = pl.program_id(0)
    n = i // NBLK

    @pl.when(i == 0)
    def _init():
        acc_i[...] = jnp.zeros_like(acc_i)
        acc_u[...] = jnp.zeros_like(acc_u)

    blk = src_ref[0]
    tgt = tgt_ref[0]
    pred = blk > 0.0
    gt = (tgt > 0.0)[None, :, :]
    inter = jnp.logical_and(pred, gt).astype(jnp.float32)
    union = jnp.logical_or(pred, gt).astype(jnp.float32)
    isum = jnp.sum(inter.reshape(M, BH * W), axis=1, keepdims=True)
    usum = jnp.sum(union.reshape(M, BH * W), axis=1, keepdims=True)
    col = (lax.broadcasted_iota(jnp.int32, (1, N), 1) == n)
    colf = col.astype(jnp.float32)
    acc_i[...] += isum * colf
    acc_u[...] += usum * colf

    @pl.when(i == GRID - 1)
    def _finish():
        sv = sv_ref[...]
        tv = tv_ref[...]

        aabs = jnp.abs(sv)
        u = -(((aabs[0] + aabs[1]) + aabs[2]) / 3.0)
        uc = u[:, :K_OVER]
        bits = lax.bitcast_convert_type(uc, jnp.uint32)
        neg = bits >= jnp.uint32(0x80000000)
        ukey = jnp.where(neg, ~bits, bits | jnp.uint32(0x80000000))

        T = jnp.zeros((N, 1), dtype=jnp.uint32)
        for bit in range(31, -1, -1):
            Tt = T | jnp.uint32(1 << bit)
            cnt = jnp.sum((ukey >= Tt).astype(jnp.int32), axis=1,
                          keepdims=True)
            T = jnp.where(cnt >= K_IMP, Tt, T)
        cnt_gt = jnp.sum((ukey > T).astype(jnp.int32), axis=1, keepdims=True)
        need = K_IMP - cnt_gt
        eq = ukey == T
        jio = lax.broadcasted_iota(jnp.int32, (N, K_OVER), 1)
        Mv = jnp.full((N, 1), 8191, dtype=jnp.int32)
        for bit in range(12, -1, -1):
            Mt = Mv & jnp.int32(~(1 << bit))
            cle = jnp.sum(jnp.logical_and(eq, jio <= Mt).astype(jnp.int32),
                          axis=1, keepdims=True)
            Mv = jnp.where(cle >= need, Mt, Mv)
        sel = jnp.logical_or(ukey > T, jnp.logical_and(eq, jio <= Mv))
        wsel = jnp.concatenate(
            [sel.astype(jnp.float32),
             jnp.ones((N, K_REM), dtype=jnp.float32)], axis=1)

        x = sv
        t = tv[None, :, :]
        wgt = wsel[None, :, :]
        prob = 1.0 / (1.0 + jnp.exp(-x))
        ce = jnp.maximum(x, 0.0) - x * t + jnp.log1p(jnp.exp(-jnp.abs(x)))
        p_t = prob * t + (1.0 - prob) * (1.0 - t)
        omp = 1.0 - p_t
        alpha_t = FOCAL_ALPHA * t + (1.0 - FOCAL_ALPHA) * (1.0 - t)
        focal = alpha_t * ce * omp * omp
        lm = jnp.sum(focal * wgt, axis=2) / (K * NUM_OBJECTS)

        num = 2.0 * jnp.sum(prob * t * wgt, axis=2)
        den = jnp.sum(prob * wgt, axis=2) + jnp.sum(t * wgt, axis=2)
        ld = (1.0 - (num + 1.0) / (den + 1.0)) / NUM_OBJECTS

        act = acc_i[...] / jnp.maximum(acc_u[...], 1.0)
        li = (piou_ref[...] - act) ** 2 / NUM_OBJECTS

        combo = lm * W_MASK + ld * W_DICE
        c0, c1, c2 = combo[0:1], combo[1:2], combo[2:3]
        b0 = jnp.logical_and(c0 <= c1, c0 <= c2)
        b1 = jnp.logical_and(jnp.logical_not(b0), c1 <= c2)

        def pick(a):
            return jnp.where(b0, a[0:1], jnp.where(b1, a[1:2], a[2:3]))

        def s11(a):
            return jnp.sum(a, axis=1, keepdims=True)

        total = (W_MASK * s11(pick(lm))
                 + W_DICE * s11(pick(ld))
                 + W_IOU * s11(pick(li)))
        out_ref[...] = total


def _tc_combine(src_masks, targets, src_vals, tgt_vals, pred_ious_t,
                interpret=False):
    return pl.pallas_call(
        _tc_body,
        grid=(GRID,),
        in_specs=[
            pl.BlockSpec((1, M, BH, W), lambda i: (i // NBLK, 0, i % NBLK, 0)),
            pl.BlockSpec((1, BH, W), lambda i: (i // NBLK, i % NBLK, 0)),
            pl.BlockSpec((M, N, K_ALL), lambda i: (0, 0, 0)),
            pl.BlockSpec((N, K_ALL), lambda i: (0, 0)),
            pl.BlockSpec((M, N), lambda i: (0, 0)),
        ],
        out_specs=pl.BlockSpec((1, 1), lambda i: (0, 0)),
        out_shape=jax.ShapeDtypeStruct((1, 1), jnp.float32),
        scratch_shapes=[
            pltpu.VMEM((M, N), jnp.float32),
            pltpu.VMEM((M, N), jnp.float32),
        ],
        interpret=interpret,
    )(src_masks, targets, src_vals, tgt_vals, pred_ious_t)


def kernel(src_masks, pred_ious, targets_batch):
    src_flat = src_masks.reshape(-1)
    tgt_flat = targets_batch.reshape(-1)
    sidx = jnp.asarray(_SRC_IDX_NP)
    tidx = jnp.asarray(_TGT_IDX_NP)
    gather = _build_sc_gather()
    src_g, tgt_g = gather(src_flat, sidx, tgt_flat, tidx)
    src_vals = src_g.reshape(M, N, K_ALL)
    tgt_vals = tgt_g.reshape(N, K_ALL)
    total = _tc_combine(
        src_masks,
        targets_batch.reshape(N, H, W),
        src_vals,
        tgt_vals,
        jnp.transpose(pred_ious),
    )
    return total[0, 0]


# speedup vs baseline: 2.5066x; 1.4502x over previous
import functools

import jax
import jax.numpy as jnp
import numpy as np
from jax import lax
from jax.experimental import pallas as pl
from jax.experimental.pallas import tpu as pltpu
from jax.experimental.pallas import tpu_sc as plsc

N, M, H, W = 16, 3, 512, 512
HW = H * W
K = 2048
K_OVER = 6144
K_IMP = 1536
K_REM = 512
K_ALL = K_OVER + K_REM
FOCAL_ALPHA = 0.25
FOCAL_GAMMA = 2.0
W_MASK = 20.0
W_DICE = 1.0
W_IOU = 1.0
NUM_OBJECTS = 16.0

SRC_TOT = M * N * K_ALL
TGT_TOT = N * K_ALL


@functools.lru_cache(maxsize=1)
def _point_indices():
    key = jax.random.key(1234)
    ky, kx, ky2, kx2 = jax.random.split(key, 4)
    y_rand = np.asarray(jax.random.randint(ky, (N, K_OVER), 0, H))
    x_rand = np.asarray(jax.random.randint(kx, (N, K_OVER), 0, W))
    y_rest = np.asarray(jax.random.randint(ky2, (N, K_REM), 0, H))
    x_rest = np.asarray(jax.random.randint(kx2, (N, K_REM), 0, W))
    lin = np.concatenate([y_rand * W + x_rand, y_rest * W + x_rest], axis=1)
    lin = lin.astype(np.int32)
    n_ids = np.arange(N, dtype=np.int32)[:, None]
    src_idx = np.empty((M, N, K_ALL), dtype=np.int32)
    for c in range(M):
        src_idx[c] = (n_ids * M + c) * HW + lin
    tgt_idx = (n_ids * HW + lin).astype(np.int32)
    return src_idx.reshape(-1), tgt_idx.reshape(-1)


_SRC_IDX_NP, _TGT_IDX_NP = _point_indices()


def _build_sc_gather():
    info = plsc.get_sparse_core_info()
    nw = info.num_cores * info.num_subcores
    src_per = SRC_TOT // nw
    tgt_per = TGT_TOT // nw
    assert SRC_TOT % nw == 0 and TGT_TOT % nw == 0
    assert src_per % 8 == 0 and tgt_per % 8 == 0
    mesh = plsc.VectorSubcoreMesh(core_axis_name="c", subcore_axis_name="s")

    @functools.partial(
        pl.kernel,
        out_type=[
            jax.ShapeDtypeStruct((SRC_TOT,), jnp.float32),
            jax.ShapeDtypeStruct((TGT_TOT,), jnp.float32),
        ],
        mesh=mesh,
        scratch_types=[
            pltpu.VMEM((src_per,), jnp.int32),
            pltpu.VMEM((src_per,), jnp.float32),
            pltpu.VMEM((tgt_per,), jnp.int32),
            pltpu.VMEM((tgt_per,), jnp.float32),
            pltpu.SemaphoreType.DMA,
            pltpu.SemaphoreType.DMA,
        ],
    )
    def gather(src_flat, src_idx, tgt_flat, tgt_idx, src_out, tgt_out,
               sidx_v, sval_v, tidx_v, tval_v, sem_s, sem_t):
        wid = lax.axis_index("s") * info.num_cores + lax.axis_index("c")
        sbase = wid * src_per
        tbase = wid * tgt_per
        pltpu.sync_copy(src_idx.at[pl.ds(sbase, src_per)], sidx_v)
        pltpu.sync_copy(tgt_idx.at[pl.ds(tbase, tgt_per)], tidx_v)
        cp_s = pltpu.async_copy(src_flat.at[sidx_v], sval_v, sem_s)
        cp_t = pltpu.async_copy(tgt_flat.at[tidx_v], tval_v, sem_t)
        cp_s.wait()
        cp_t.wait()
        pltpu.sync_copy(sval_v, src_out.at[pl.ds(sbase, src_per)])
        pltpu.sync_copy(tval_v, tgt_out.at[pl.ds(tbase, tgt_per)])

    return gather


def _iou_body(src_ref, tgt_ref, out_i, out_u):
    i = pl.program_id(0)

    @pl.when(i == 0)
    def _init():
        out_i[...] = jnp.zeros_like(out_i)
        out_u[...] = jnp.zeros_like(out_u)

    blk = src_ref[...]
    tgt = tgt_ref[...]
    pred = blk > 0.0
    gt = (tgt > 0.0)[:, None, :, :]
    inter = jnp.logical_and(pred, gt).astype(jnp.float32)
    union = jnp.logical_or(pred, gt).astype(jnp.float32)
    isum = jnp.sum(jnp.sum(inter, axis=(0, 3)), axis=1, keepdims=True)
    usum = jnp.sum(jnp.sum(union, axis=(0, 3)), axis=1, keepdims=True)
    colf = (lax.broadcasted_iota(jnp.int32, (1, N), 1) == i).astype(jnp.float32)
    out_i[...] += isum * colf
    out_u[...] += usum * colf


def _iou_counts(src_masks, targets):
    return pl.pallas_call(
        _iou_body,
        grid=(N,),
        in_specs=[
            pl.BlockSpec((1, M, H, W), lambda i: (i, 0, 0, 0)),
            pl.BlockSpec((1, H, W), lambda i: (i, 0, 0)),
        ],
        out_specs=[
            pl.BlockSpec((M, N), lambda i: (0, 0)),
            pl.BlockSpec((M, N), lambda i: (0, 0)),
        ],
        out_shape=[
            jax.ShapeDtypeStruct((M, N), jnp.float32),
            jax.ShapeDtypeStruct((M, N), jnp.float32),
        ],
    )(src_masks, targets)


def _combine_body(sv_ref, tv_ref, piou_ref, acci_ref, accu_ref, out_ref):
    if True:
        sv = sv_ref[...]
        tv = tv_ref[...]

        aabs = jnp.abs(sv)
        u = -(((aabs[0] + aabs[1]) + aabs[2]) / 3.0)
        uc = u[:, :K_OVER]
        bits = lax.bitcast_convert_type(uc, jnp.uint32)
        neg = bits >= jnp.uint32(0x80000000)
        ukey = jnp.where(neg, ~bits, bits | jnp.uint32(0x80000000))

        T = jnp.zeros((N, 1), dtype=jnp.uint32)
        for bit in range(31, -1, -1):
            Tt = T | jnp.uint32(1 << bit)
            cnt = jnp.sum((ukey >= Tt).astype(jnp.int32), axis=1,
                          keepdims=True)
            T = jnp.where(cnt >= K_IMP, Tt, T)
        cnt_gt = jnp.sum((ukey > T).astype(jnp.int32), axis=1, keepdims=True)
        need = K_IMP - cnt_gt
        eq = ukey == T
        jio = lax.broadcasted_iota(jnp.int32, (N, K_OVER), 1)
        Mv = jnp.full((N, 1), 8191, dtype=jnp.int32)
        for bit in range(12, -1, -1):
            Mt = Mv & jnp.int32(~(1 << bit))
            cle = jnp.sum(jnp.logical_and(eq, jio <= Mt).astype(jnp.int32),
                          axis=1, keepdims=True)
            Mv = jnp.where(cle >= need, Mt, Mv)
        sel = jnp.logical_or(ukey > T, jnp.logical_and(eq, jio <= Mv))
        wsel = jnp.concatenate(
            [sel.astype(jnp.float32),
             jnp.ones((N, K_REM), dtype=jnp.float32)], axis=1)

        x = sv
        t = tv[None, :, :]
        wgt = wsel[None, :, :]
        prob = 1.0 / (1.0 + jnp.exp(-x))
        ce = jnp.maximum(x, 0.0) - x * t + jnp.log1p(jnp.exp(-jnp.abs(x)))
        p_t = prob * t + (1.0 - prob) * (1.0 - t)
        omp = 1.0 - p_t
        alpha_t = FOCAL_ALPHA * t + (1.0 - FOCAL_ALPHA) * (1.0 - t)
        focal = alpha_t * ce * omp * omp
        lm = jnp.sum(focal * wgt, axis=2) / (K * NUM_OBJECTS)

        num = 2.0 * jnp.sum(prob * t * wgt, axis=2)
        den = jnp.sum(prob * wgt, axis=2) + jnp.sum(t * wgt, axis=2)
        ld = (1.0 - (num + 1.0) / (den + 1.0)) / NUM_OBJECTS

        act = acci_ref[...] / jnp.maximum(accu_ref[...], 1.0)
        li = (piou_ref[...] - act) ** 2 / NUM_OBJECTS

        combo = lm * W_MASK + ld * W_DICE
        c0, c1, c2 = combo[0:1], combo[1:2], combo[2:3]
        b0 = jnp.logical_and(c0 <= c1, c0 <= c2)
        b1 = jnp.logical_and(jnp.logical_not(b0), c1 <= c2)

        def pick(a):
            return jnp.where(b0, a[0:1], jnp.where(b1, a[1:2], a[2:3]))

        def s11(a):
            return jnp.sum(a, axis=1, keepdims=True)

        total = (W_MASK * s11(pick(lm))
                 + W_DICE * s11(pick(ld))
                 + W_IOU * s11(pick(li)))
        out_ref[...] = total


def _tc_combine(src_vals, tgt_vals, pred_ious_t, acc_i, acc_u,
                interpret=False):
    return pl.pallas_call(
        _combine_body,
        in_specs=[
            pl.BlockSpec((M, N, K_ALL), lambda: (0, 0, 0)),
            pl.BlockSpec((N, K_ALL), lambda: (0, 0)),
            pl.BlockSpec((M, N), lambda: (0, 0)),
            pl.BlockSpec((M, N), lambda: (0, 0)),
            pl.BlockSpec((M, N), lambda: (0, 0)),
        ],
        out_specs=pl.BlockSpec((1, 1), lambda: (0, 0)),
        out_shape=jax.ShapeDtypeStruct((1, 1), jnp.float32),
        interpret=interpret,
    )(src_vals, tgt_vals, pred_ious_t, acc_i, acc_u)


def kernel(src_masks, pred_ious, targets_batch):
    src_flat = src_masks.reshape(-1)
    tgt_flat = targets_batch.reshape(-1)
    sidx = jnp.asarray(_SRC_IDX_NP)
    tidx = jnp.asarray(_TGT_IDX_NP)
    acc_i, acc_u = _iou_counts(src_masks, targets_batch.reshape(N, H, W))
    gather = _build_sc_gather()
    src_g, tgt_g = gather(src_flat, sidx, tgt_flat, tidx)
    src_vals = src_g.reshape(M, N, K_ALL)
    tgt_vals = tgt_g.reshape(N, K_ALL)
    total = _tc_combine(
        src_vals,
        tgt_vals,
        jnp.transpose(pred_ious),
        acc_i,
        acc_u,
    )
    return total[0, 0]
